# baseline (speedup 1.0000x reference)
import jax
import jax.numpy as jnp
from jax.experimental import pallas as pl


def _deg_body(adj_ref, deg_ref, adjb_ref):
    a = adj_ref[:]
    deg_ref[:] = jnp.sum(a, axis=1, keepdims=True)
    adjb_ref[:] = a.astype(jnp.bfloat16)


def _x1_body(adj_ref, xs_ref, d_ref, o_ref):
    o_ref[:] = -d_ref[:] * jnp.dot(adj_ref[:], xs_ref[:],
                                   preferred_element_type=jnp.float32)


def _supp_body(adj_ref, y1_ref, x0_ref, x1_ref, d_ref,
               w0_ref, w1_ref, w2_ref, bc_ref, wo_ref, o_ref):
    x2 = (-2.0 * d_ref[:] * jnp.dot(adj_ref[:], y1_ref[:],
                                    preferred_element_type=jnp.float32)
          - x0_ref[:])
    h = (jnp.dot(x0_ref[:], w0_ref[:], preferred_element_type=jnp.float32)
         + jnp.dot(x1_ref[:], w1_ref[:], preferred_element_type=jnp.float32)
         + jnp.dot(x2, w2_ref[:], preferred_element_type=jnp.float32)
         + bc_ref[:])
    h = jnp.maximum(h, 0.0)
    o_ref[:] = jnp.dot(h, wo_ref[:], preferred_element_type=jnp.float32)


def _pool_body(adj_ref, s_ref, o_ref):
    i = pl.program_id(0)
    part = jnp.dot(adj_ref[:], s_ref[:], preferred_element_type=jnp.float32)
    m = jnp.max(part, axis=0, keepdims=True)

    @pl.when(i == 0)
    def _init():
        o_ref[:] = m

    @pl.when(i != 0)
    def _acc():
        o_ref[:] = jnp.maximum(o_ref[:], m)


def kernel(x, adj, W_cheb, b_cheb, W2, b2):
    N, F = x.shape
    H = W_cheb.shape[2]
    C = W2.shape[1]
    R = next((r for r in (200, 80, 40, 16, 8) if N % r == 0), N)
    G = N // R

    deg, adjb = pl.pallas_call(
        _deg_body,
        grid=(G,),
        in_specs=[pl.BlockSpec((R, N), lambda i: (i, 0))],
        out_specs=[pl.BlockSpec((R, 1), lambda i: (i, 0)),
                   pl.BlockSpec((R, N), lambda i: (i, 0))],
        out_shape=[jax.ShapeDtypeStruct((N, 1), jnp.float32),
                   jax.ShapeDtypeStruct((N, N), jnp.bfloat16)],
    )(adj)

    d = jnp.where(deg > 0, jax.lax.rsqrt(jnp.maximum(deg, 1e-12)), 0.0)
    xs = (x * d).astype(jnp.bfloat16)

    X1 = pl.pallas_call(
        _x1_body,
        grid=(G,),
        in_specs=[
            pl.BlockSpec((R, N), lambda i: (i, 0)),
            pl.BlockSpec((N, F), lambda i: (0, 0)),
            pl.BlockSpec((R, 1), lambda i: (i, 0)),
        ],
        out_specs=pl.BlockSpec((R, F), lambda i: (i, 0)),
        out_shape=jax.ShapeDtypeStruct((N, F), jnp.float32),
    )(adjb, xs, d)

    y1 = (X1 * d).astype(jnp.bfloat16)

    support = pl.pallas_call(
        _supp_body,
        grid=(G,),
        in_specs=[
            pl.BlockSpec((R, N), lambda i: (i, 0)),
            pl.BlockSpec((N, F), lambda i: (0, 0)),
            pl.BlockSpec((R, F), lambda i: (i, 0)),
            pl.BlockSpec((R, F), lambda i: (i, 0)),
            pl.BlockSpec((R, 1), lambda i: (i, 0)),
            pl.BlockSpec((F, H), lambda i: (0, 0)),
            pl.BlockSpec((F, H), lambda i: (0, 0)),
            pl.BlockSpec((F, H), lambda i: (0, 0)),
            pl.BlockSpec((1, H), lambda i: (0, 0)),
            pl.BlockSpec((H, C), lambda i: (0, 0)),
        ],
        out_specs=pl.BlockSpec((R, C), lambda i: (i, 0)),
        out_shape=jax.ShapeDtypeStruct((N, C), jnp.float32),
    )(adjb, y1, x, X1, d, W_cheb[0], W_cheb[1], W_cheb[2],
      b_cheb.reshape(1, H), W2)

    pooled = pl.pallas_call(
        _pool_body,
        grid=(G,),
        in_specs=[
            pl.BlockSpec((R, N), lambda i: (i, 0)),
            pl.BlockSpec((N, C), lambda i: (0, 0)),
        ],
        out_specs=pl.BlockSpec((1, C), lambda i: (0, 0)),
        out_shape=jax.ShapeDtypeStruct((1, C), jnp.float32),
    )(adjb, support.astype(jnp.bfloat16))

    return (pooled + b2)[None, :, :]


# speedup vs baseline: 1.1250x; 1.1250x over previous
import jax
import jax.numpy as jnp
from jax.experimental import pallas as pl


def _deg_body(adj_ref, deg_ref, adjb_ref):
    a = adj_ref[:]
    deg_ref[:] = jnp.sum(a, axis=1, keepdims=True)
    adjb_ref[:] = a.astype(jnp.bfloat16)


def _x1_body(adj_ref, xs_ref, d_ref, o_ref, y1_ref):
    x1 = -d_ref[:] * jnp.dot(adj_ref[:], xs_ref[:],
                             preferred_element_type=jnp.float32)
    o_ref[:] = x1
    y1_ref[:] = (d_ref[:] * x1).astype(jnp.bfloat16)


def _supp_body(adj_ref, y1_ref, x0_ref, x1_ref, d_ref,
               w0_ref, w1_ref, w2_ref, bc_ref, wo_ref, o_ref):
    x2 = (-2.0 * d_ref[:] * jnp.dot(adj_ref[:], y1_ref[:],
                                    preferred_element_type=jnp.float32)
          - x0_ref[:])
    h = (jnp.dot(x0_ref[:], w0_ref[:], preferred_element_type=jnp.float32)
         + jnp.dot(x1_ref[:], w1_ref[:], preferred_element_type=jnp.float32)
         + jnp.dot(x2, w2_ref[:], preferred_element_type=jnp.float32)
         + bc_ref[:])
    h = jnp.maximum(h, 0.0)
    o_ref[:] = jnp.dot(h, wo_ref[:],
                       preferred_element_type=jnp.float32).astype(jnp.bfloat16)


def _pool_body(adj_ref, s_ref, o_ref):
    i = pl.program_id(0)
    part = jnp.dot(adj_ref[:], s_ref[:], preferred_element_type=jnp.float32)
    m = jnp.max(part, axis=0, keepdims=True)

    @pl.when(i == 0)
    def _init():
        o_ref[:] = m

    @pl.when(i != 0)
    def _acc():
        o_ref[:] = jnp.maximum(o_ref[:], m)


def kernel(x, adj, W_cheb, b_cheb, W2, b2):
    N, F = x.shape
    H = W_cheb.shape[2]
    C = W2.shape[1]
    R = next((r for r in (400, 200, 80, 40, 16, 8) if N % r == 0), N)
    G = N // R

    deg, adjb = pl.pallas_call(
        _deg_body,
        grid=(G,),
        in_specs=[pl.BlockSpec((R, N), lambda i: (i, 0))],
        out_specs=[pl.BlockSpec((R, 1), lambda i: (i, 0)),
                   pl.BlockSpec((R, N), lambda i: (i, 0))],
        out_shape=[jax.ShapeDtypeStruct((N, 1), jnp.float32),
                   jax.ShapeDtypeStruct((N, N), jnp.bfloat16)],
    )(adj)

    d = jnp.where(deg > 0, jax.lax.rsqrt(jnp.maximum(deg, 1e-12)), 0.0)
    xs = (x * d).astype(jnp.bfloat16)

    X1, y1 = pl.pallas_call(
        _x1_body,
        grid=(G,),
        in_specs=[
            pl.BlockSpec((R, N), lambda i: (i, 0)),
            pl.BlockSpec((N, F), lambda i: (0, 0)),
            pl.BlockSpec((R, 1), lambda i: (i, 0)),
        ],
        out_specs=[pl.BlockSpec((R, F), lambda i: (i, 0)),
                   pl.BlockSpec((R, F), lambda i: (i, 0))],
        out_shape=[jax.ShapeDtypeStruct((N, F), jnp.float32),
                   jax.ShapeDtypeStruct((N, F), jnp.bfloat16)],
    )(adjb, xs, d)

    support = pl.pallas_call(
        _supp_body,
        grid=(G,),
        in_specs=[
            pl.BlockSpec((R, N), lambda i: (i, 0)),
            pl.BlockSpec((N, F), lambda i: (0, 0)),
            pl.BlockSpec((R, F), lambda i: (i, 0)),
            pl.BlockSpec((R, F), lambda i: (i, 0)),
            pl.BlockSpec((R, 1), lambda i: (i, 0)),
            pl.BlockSpec((F, H), lambda i: (0, 0)),
            pl.BlockSpec((F, H), lambda i: (0, 0)),
            pl.BlockSpec((F, H), lambda i: (0, 0)),
            pl.BlockSpec((1, H), lambda i: (0, 0)),
            pl.BlockSpec((H, C), lambda i: (0, 0)),
        ],
        out_specs=pl.BlockSpec((R, C), lambda i: (i, 0)),
        out_shape=jax.ShapeDtypeStruct((N, C), jnp.bfloat16),
    )(adjb, y1, x, X1, d, W_cheb[0], W_cheb[1], W_cheb[2],
      b_cheb.reshape(1, H), W2)

    pooled = pl.pallas_call(
        _pool_body,
        grid=(G,),
        in_specs=[
            pl.BlockSpec((R, N), lambda i: (i, 0)),
            pl.BlockSpec((N, C), lambda i: (0, 0)),
        ],
        out_specs=pl.BlockSpec((1, C), lambda i: (0, 0)),
        out_shape=jax.ShapeDtypeStruct((1, C), jnp.float32),
    )(adjb, support)

    return (pooled + b2)[None, :, :]


# speedup vs baseline: 1.1562x; 1.0277x over previous
import jax
import jax.numpy as jnp
from jax.experimental import pallas as pl
from jax.experimental.pallas import tpu as pltpu


def _deg_body(adj_ref, deg_ref, adjb_ref):
    a = adj_ref[:]
    deg_ref[:] = jnp.sum(a, axis=1, keepdims=True)
    adjb_ref[:] = a.astype(jnp.bfloat16)


def _x1_body(adj_ref, xs_ref, d_ref, y1_ref):
    x1 = -d_ref[:] * jnp.dot(adj_ref[:], xs_ref[:],
                             preferred_element_type=jnp.float32)
    y1_ref[:] = (d_ref[:] * x1).astype(jnp.bfloat16)


def _supp_body(adjr_ref, y1_ref, x0_ref, d_ref, dinv_ref,
               w0_ref, w1_ref, w2_ref, bc_ref, wo_ref, s_ref):
    i = pl.program_id(0)
    r = x0_ref.shape[0]
    x0 = x0_ref[:]
    x2 = (-2.0 * d_ref[:] * jnp.dot(adjr_ref[:], y1_ref[:],
                                    preferred_element_type=jnp.float32)
          - x0)
    y1_blk = y1_ref[pl.ds(i * r, r), :].astype(jnp.float32)
    h = (jnp.dot(x0, w0_ref[:], preferred_element_type=jnp.float32)
         + dinv_ref[:] * jnp.dot(y1_blk, w1_ref[:],
                                 preferred_element_type=jnp.float32)
         + jnp.dot(x2, w2_ref[:], preferred_element_type=jnp.float32)
         + bc_ref[:])
    h = jnp.maximum(h, 0.0)
    s_ref[:] = jnp.dot(h, wo_ref[:],
                       preferred_element_type=jnp.float32).astype(jnp.bfloat16)


def _pool_body(adj_ref, s_ref, b2_ref, o_ref):
    i = pl.program_id(0)
    part = jnp.dot(adj_ref[:], s_ref[:], preferred_element_type=jnp.float32)
    m = jnp.max(part, axis=0, keepdims=True)

    @pl.when(i == 0)
    def _init():
        o_ref[:] = m + b2_ref[:]

    @pl.when(i != 0)
    def _acc():
        o_ref[:] = jnp.maximum(o_ref[:], m + b2_ref[:])


def kernel(x, adj, W_cheb, b_cheb, W2, b2):
    N, F = x.shape
    H = W_cheb.shape[2]
    C = W2.shape[1]
    R1 = next((r for r in (400, 200, 80, 40, 16, 8) if N % r == 0), N)
    R2 = next((r for r in (1000, 400, 200, 80, 40, 16, 8) if N % r == 0), N)
    R3 = next((r for r in (1000, 400, 200, 80, 40, 16, 8) if N % r == 0), N)

    deg, adjb = pl.pallas_call(
        _deg_body,
        grid=(N // R1,),
        in_specs=[pl.BlockSpec((R1, N), lambda i: (i, 0))],
        out_specs=[pl.BlockSpec((R1, 1), lambda i: (i, 0)),
                   pl.BlockSpec((R1, N), lambda i: (i, 0))],
        out_shape=[jax.ShapeDtypeStruct((N, 1), jnp.float32),
                   jax.ShapeDtypeStruct((N, N), jnp.bfloat16)],
    )(adj)

    d = jnp.where(deg > 0, jax.lax.rsqrt(jnp.maximum(deg, 1e-12)), 0.0)
    dinv = jnp.where(deg > 0, jnp.sqrt(jnp.maximum(deg, 1e-12)), 0.0)
    xs = (x * d).astype(jnp.bfloat16)

    y1 = pl.pallas_call(
        _x1_body,
        grid=(N // R2,),
        in_specs=[
            pl.BlockSpec((R2, N), lambda i: (i, 0)),
            pl.BlockSpec((N, F), lambda i: (0, 0)),
            pl.BlockSpec((R2, 1), lambda i: (i, 0)),
        ],
        out_specs=pl.BlockSpec((R2, F), lambda i: (i, 0)),
        out_shape=jax.ShapeDtypeStruct((N, F), jnp.bfloat16),
    )(adjb, xs, d)

    support = pl.pallas_call(
        _supp_body,
        grid=(N // R3,),
        in_specs=[
            pl.BlockSpec((R3, N), lambda i: (i, 0)),
            pl.BlockSpec((N, F), lambda i: (0, 0)),
            pl.BlockSpec((R3, F), lambda i: (i, 0)),
            pl.BlockSpec((R3, 1), lambda i: (i, 0)),
            pl.BlockSpec((R3, 1), lambda i: (i, 0)),
            pl.BlockSpec((F, H), lambda i: (0, 0)),
            pl.BlockSpec((F, H), lambda i: (0, 0)),
            pl.BlockSpec((F, H), lambda i: (0, 0)),
            pl.BlockSpec((1, H), lambda i: (0, 0)),
            pl.BlockSpec((H, C), lambda i: (0, 0)),
        ],
        out_specs=pl.BlockSpec((R3, C), lambda i: (i, 0)),
        out_shape=jax.ShapeDtypeStruct((N, C), jnp.bfloat16),
    )(adjb, y1, x, d, dinv, W_cheb[0], W_cheb[1], W_cheb[2],
      b_cheb.reshape(1, H), W2)

    pooled = pl.pallas_call(
        _pool_body,
        grid=(N // R2,),
        in_specs=[
            pl.BlockSpec((R2, N), lambda i: (i, 0)),
            pl.BlockSpec((N, C), lambda i: (0, 0)),
            pl.BlockSpec((1, C), lambda i: (0, 0)),
        ],
        out_specs=pl.BlockSpec((1, C), lambda i: (0, 0)),
        out_shape=jax.ShapeDtypeStruct((1, C), jnp.float32),
    )(adjb, support, b2.reshape(1, C))

    return pooled[None, :, :]


# speedup vs baseline: 1.2118x; 1.0481x over previous
import jax
import jax.numpy as jnp
from jax.experimental import pallas as pl
from jax.experimental.pallas import tpu as pltpu


def _deg_body(adj_ref, deg_ref, adjb_ref):
    a = adj_ref[:]
    deg_ref[:] = jnp.sum(a, axis=1, keepdims=True)
    adjb_ref[:] = a.astype(jnp.bfloat16)


def _main_body(adjb_ref, xs_ref, x0_ref, d_ref, dinv_ref,
               w0_ref, w1_ref, w2_ref, bc_ref, wo_ref, b2_ref,
               o_ref, y1_scr, s_scr):
    p = pl.program_id(0)
    i = pl.program_id(1)
    r = adjb_ref.shape[0]

    @pl.when(p == 0)
    def _phase_y1():
        x1 = -d_ref[:] * jnp.dot(adjb_ref[:], xs_ref[:],
                                 preferred_element_type=jnp.float32)
        y1_scr[pl.ds(i * r, r), :] = (d_ref[:] * x1).astype(jnp.bfloat16)

    @pl.when(p == 1)
    def _phase_support():
        x0 = x0_ref[:]
        x2 = (-2.0 * d_ref[:] * jnp.dot(adjb_ref[:], y1_scr[:],
                                        preferred_element_type=jnp.float32)
              - x0)
        y1_blk = y1_scr[pl.ds(i * r, r), :].astype(jnp.float32)
        h = (jnp.dot(x0, w0_ref[:], preferred_element_type=jnp.float32)
             + dinv_ref[:] * jnp.dot(y1_blk, w1_ref[:],
                                     preferred_element_type=jnp.float32)
             + jnp.dot(x2, w2_ref[:], preferred_element_type=jnp.float32)
             + bc_ref[:])
        h = jnp.maximum(h, 0.0)
        s_scr[pl.ds(i * r, r), :] = jnp.dot(
            h, wo_ref[:], preferred_element_type=jnp.float32
        ).astype(jnp.bfloat16)

    @pl.when(p == 2)
    def _phase_pool():
        part = jnp.dot(adjb_ref[:], s_scr[:],
                       preferred_element_type=jnp.float32)
        m = jnp.max(part, axis=0, keepdims=True) + b2_ref[:]

        @pl.when(i == 0)
        def _init():
            o_ref[:] = m

        @pl.when(i != 0)
        def _acc():
            o_ref[:] = jnp.maximum(o_ref[:], m)


def kernel(x, adj, W_cheb, b_cheb, W2, b2):
    N, F = x.shape
    H = W_cheb.shape[2]
    C = W2.shape[1]
    R1 = next((r for r in (400, 200, 80, 40, 16, 8) if N % r == 0), N)
    R2 = next((r for r in (1000, 400, 200, 80, 40, 16, 8) if N % r == 0), N)

    deg, adjb = pl.pallas_call(
        _deg_body,
        grid=(N // R1,),
        in_specs=[pl.BlockSpec((R1, N), lambda i: (i, 0))],
        out_specs=[pl.BlockSpec((R1, 1), lambda i: (i, 0)),
                   pl.BlockSpec((R1, N), lambda i: (i, 0))],
        out_shape=[jax.ShapeDtypeStruct((N, 1), jnp.float32),
                   jax.ShapeDtypeStruct((N, N), jnp.bfloat16)],
    )(adj)

    d = jnp.where(deg > 0, jax.lax.rsqrt(jnp.maximum(deg, 1e-12)), 0.0)
    dinv = jnp.where(deg > 0, jnp.sqrt(jnp.maximum(deg, 1e-12)), 0.0)
    xs = (x * d).astype(jnp.bfloat16)

    pooled = pl.pallas_call(
        _main_body,
        grid=(3, N // R2),
        in_specs=[
            pl.BlockSpec((R2, N), lambda p, i: (i, 0)),
            pl.BlockSpec((N, F), lambda p, i: (0, 0)),
            pl.BlockSpec((R2, F), lambda p, i: (jnp.where(p == 1, i, 0), 0)),
            pl.BlockSpec((R2, 1), lambda p, i: (i, 0)),
            pl.BlockSpec((R2, 1), lambda p, i: (i, 0)),
            pl.BlockSpec((F, H), lambda p, i: (0, 0)),
            pl.BlockSpec((F, H), lambda p, i: (0, 0)),
            pl.BlockSpec((F, H), lambda p, i: (0, 0)),
            pl.BlockSpec((1, H), lambda p, i: (0, 0)),
            pl.BlockSpec((H, C), lambda p, i: (0, 0)),
            pl.BlockSpec((1, C), lambda p, i: (0, 0)),
        ],
        out_specs=pl.BlockSpec((1, C), lambda p, i: (0, 0)),
        out_shape=jax.ShapeDtypeStruct((1, C), jnp.float32),
        scratch_shapes=[pltpu.VMEM((N, F), jnp.bfloat16),
                        pltpu.VMEM((N, C), jnp.bfloat16)],
    )(adjb, xs, x, d, dinv, W_cheb[0], W_cheb[1], W_cheb[2],
      b_cheb.reshape(1, H), W2, b2.reshape(1, C))

    return pooled[None, :, :]
